# baseline (speedup 1.0000x reference)
import functools

import jax
import jax.numpy as jnp
from jax import lax
from jax.experimental import pallas as pl
from jax.experimental.pallas import tpu as pltpu
from jax.experimental.pallas import tpu_sc as plsc

B, S = 8, 2048
EMB_W, HIDDEN = 128, 768
BS = B * S


_SC_CHUNK = 128


def _sc_gather_build():
    info = plsc.get_sparse_core_info()
    nc, ns = info.num_cores, info.num_subcores
    nw = nc * ns
    rows_per_w = BS // nw
    rows_per_core = BS // nc
    b_per_core = B // nc
    n_chunks = rows_per_w // _SC_CHUNK
    n_vec = S // 16

    @functools.partial(
        pl.kernel,
        mesh=plsc.VectorSubcoreMesh(core_axis_name="c", subcore_axis_name="s"),
        out_type=[
            jax.ShapeDtypeStruct((BS, EMB_W), jnp.float32),
            jax.ShapeDtypeStruct((BS, EMB_W), jnp.float32),
            jax.ShapeDtypeStruct((BS,), jnp.int32),
            jax.ShapeDtypeStruct((BS,), jnp.int32),
        ],
        scratch_types=[
            pltpu.VMEM((S,), jnp.int32),
            pltpu.VMEM((S,), jnp.int32),
            pltpu.VMEM((S,), jnp.int32),
            pltpu.VMEM((rows_per_w,), jnp.int32),
            pltpu.VMEM((rows_per_w,), jnp.int32),
            pltpu.VMEM((_SC_CHUNK, EMB_W), jnp.float32),
            pltpu.VMEM((_SC_CHUNK, EMB_W), jnp.float32),
            pltpu.SemaphoreType.DMA,
            pltpu.SemaphoreType.DMA,
        ],
    )
    def sc_gather(wtab, ptab, wids, we_out, pe_out, lab_out, pid_out,
                  w_v, lab_v, pos_v, widx_v, pidx_v, rows0, rows1,
                  sem0, sem1):
        cid = lax.axis_index("c")
        sid = lax.axis_index("s")
        base = cid * rows_per_core + sid * rows_per_w

        @pl.when(sid < b_per_core)
        def _scan_on_sc():
            rb = (cid * b_per_core + sid) * S
            pltpu.sync_copy(wids.at[pl.ds(rb, S)], w_v)
            io0 = lax.iota(jnp.int32, 16)

            def vtake(v, idx):
                return lax.gather(
                    v, idx.reshape(16, 1),
                    lax.GatherDimensionNumbers(
                        offset_dims=(), collapsed_slice_dims=(0,),
                        start_index_map=(0,)),
                    (1,), mode=lax.GatherScatterMode.PROMISE_IN_BOUNDS)

            def splat(v, i):
                return vtake(v, jnp.full((16,), i, jnp.int32))

            w0 = splat(w_v[pl.ds(0, 16)], 0)

            def body(i, m_run):
                chunk = w_v[pl.ds(i * 16, 16)]
                io = io0 + i * 16
                x = jnp.where(chunk == w0, io, jnp.zeros((16,), jnp.int32))
                for k in (1, 2, 4, 8):
                    x = jnp.maximum(x, vtake(x, jnp.maximum(io0 - k, 0)))
                mch = jnp.maximum(x, m_run)
                lab_v[pl.ds(i * 16, 16)] = mch
                pos_v[pl.ds(i * 16, 16)] = io - mch
                return splat(mch, 15)

            lax.fori_loop(0, n_vec, body, jnp.zeros((16,), jnp.int32))
            pltpu.sync_copy(lab_v, lab_out.at[pl.ds(rb, S)])
            pltpu.sync_copy(pos_v, pid_out.at[pl.ds(rb, S)])

        pltpu.sync_copy(wids.at[pl.ds(base, rows_per_w)], widx_v)
        bufs = (rows0, rows1)
        sems = (sem0, sem1)
        copies = []
        for c in range(n_chunks):
            copies.append(pltpu.async_copy(
                wtab.at[widx_v.at[pl.ds(c * _SC_CHUNK, _SC_CHUNK)]],
                bufs[c % 2], sems[c % 2]))
            if c >= 1:
                copies[c - 1].wait()
                pltpu.sync_copy(bufs[(c - 1) % 2],
                                we_out.at[pl.ds(base + (c - 1) * _SC_CHUNK,
                                                _SC_CHUNK)])
        copies[-1].wait()
        pltpu.sync_copy(bufs[(n_chunks - 1) % 2],
                        we_out.at[pl.ds(base + (n_chunks - 1) * _SC_CHUNK,
                                        _SC_CHUNK)])

        plsc.subcore_barrier()
        pltpu.sync_copy(pid_out.at[pl.ds(base, rows_per_w)], pidx_v)
        copies2 = []
        for c in range(n_chunks):
            copies2.append(pltpu.async_copy(
                ptab.at[pidx_v.at[pl.ds(c * _SC_CHUNK, _SC_CHUNK)]],
                bufs[c % 2], sems[c % 2]))
            if c >= 1:
                copies2[c - 1].wait()
                pltpu.sync_copy(bufs[(c - 1) % 2],
                                pe_out.at[pl.ds(base + (c - 1) * _SC_CHUNK,
                                                _SC_CHUNK)])
        copies2[-1].wait()
        pltpu.sync_copy(bufs[(n_chunks - 1) % 2],
                        pe_out.at[pl.ds(base + (n_chunks - 1) * _SC_CHUNK,
                                        _SC_CHUNK)])

    return sc_gather


_RA = 256
_CA = 1024


def _fused_body(seqc_ref, seqr_ref, mask_ref, we_ref, pe_ref, tid_ref,
                temb_ref, g_ref, bt_ref, proj_ref,
                attn_ref, emb_ref):
    x = we_ref[0] + pe_ref[0]
    t = tid_ref[0].astype(jnp.float32)
    t0 = temb_ref[0:1, :]
    t1 = temb_ref[1:2, :]
    x = x + t0 + t * (t1 - t0)
    mean = jnp.mean(x, axis=1, keepdims=True)
    xc = x - mean
    var = jnp.mean(xc * xc, axis=1, keepdims=True)
    y = xc * lax.rsqrt(var + 1e-12) * g_ref[...] + bt_ref[...]
    emb_ref[0] = jnp.dot(y, proj_ref[...], preferred_element_type=jnp.float32)
    sc = seqc_ref[...]
    sr = seqr_ref[...]
    mk = mask_ref[...].astype(jnp.float32)
    for c in range(S // _CA):
        lo, hi = c * _CA, (c + 1) * _CA
        eq = (sc == sr[:, :, lo:hi]).astype(jnp.float32)
        attn_ref[:, :, lo:hi] = eq * mk[:, :, lo:hi]


def _run_fused(seq_ids, input_mask, we, pe, input_type_ids,
               type_emb, ln_gamma, ln_beta, proj_kernel):
    return pl.pallas_call(
        _fused_body,
        grid=(B, S // _RA),
        in_specs=[
            pl.BlockSpec((1, _RA, 1), lambda b, j: (b, j, 0)),
            pl.BlockSpec((1, 1, S), lambda b, j: (b, 0, 0)),
            pl.BlockSpec((1, 1, S), lambda b, j: (b, 0, 0)),
            pl.BlockSpec((1, _RA, EMB_W), lambda b, j: (b, j, 0)),
            pl.BlockSpec((1, _RA, EMB_W), lambda b, j: (b, j, 0)),
            pl.BlockSpec((1, _RA, 1), lambda b, j: (b, j, 0)),
            pl.BlockSpec((2, EMB_W), lambda b, j: (0, 0)),
            pl.BlockSpec((1, EMB_W), lambda b, j: (0, 0)),
            pl.BlockSpec((1, EMB_W), lambda b, j: (0, 0)),
            pl.BlockSpec((EMB_W, HIDDEN), lambda b, j: (0, 0)),
        ],
        out_specs=[
            pl.BlockSpec((1, _RA, S), lambda b, j: (b, j, 0)),
            pl.BlockSpec((1, _RA, HIDDEN), lambda b, j: (b, j, 0)),
        ],
        out_shape=[
            jax.ShapeDtypeStruct((B, S, S), jnp.float32),
            jax.ShapeDtypeStruct((B, S, HIDDEN), jnp.float32),
        ],
    )(seq_ids.reshape(B, S, 1), seq_ids.reshape(B, 1, S),
      input_mask.reshape(B, 1, S), we.reshape(B, S, EMB_W),
      pe.reshape(B, S, EMB_W), input_type_ids.reshape(B, S, 1),
      type_emb, ln_gamma.reshape(1, EMB_W), ln_beta.reshape(1, EMB_W),
      proj_kernel)


def kernel(input_word_ids, input_mask, input_type_ids, word_emb, type_emb,
           pos_emb, ln_gamma, ln_beta, proj_kernel):
    we, pe, lab, _pid = _sc_gather_build()(word_emb, pos_emb,
                                           input_word_ids.reshape(BS))
    attn, emb = _run_fused(lab.reshape(B, S), input_mask, we, pe,
                           input_type_ids, type_emb, ln_gamma, ln_beta,
                           proj_kernel)
    return emb, attn
    we, pe = _sc_gather_build()(
        word_emb, pos_emb,
        input_word_ids.reshape(BS), pos_ids.reshape(BS))
    attn = _run_attn(seq_ids, input_mask)
    emb = _run_emb(we, pe, input_type_ids, type_emb, ln_gamma, ln_beta,
                   proj_kernel)
    return emb.reshape(B, S, HIDDEN), attn


# speedup vs baseline: 1.2262x; 1.2262x over previous
import functools

import jax
import jax.numpy as jnp
from jax import lax
from jax.experimental import pallas as pl
from jax.experimental.pallas import tpu as pltpu
from jax.experimental.pallas import tpu_sc as plsc

B, S = 8, 2048
EMB_W, HIDDEN = 128, 768
BS = B * S


def _scan_body(wid_ref, seq_ref, pos_ref):
    w = wid_ref[...]
    start = (w == w[:, 0:1]).astype(jnp.int32)
    iota = lax.broadcasted_iota(jnp.int32, (B, S), 1)
    m = iota * start
    k = 1
    while k < S:
        m = jnp.maximum(m, jnp.concatenate(
            [jnp.zeros((B, k), jnp.int32), m[:, : S - k]], axis=1))
        k *= 2
    seq_ref[...] = m
    pos_ref[...] = iota - m


def _run_scan(input_word_ids):
    return pl.pallas_call(
        _scan_body,
        out_shape=(
            jax.ShapeDtypeStruct((B, S), jnp.int32),
            jax.ShapeDtypeStruct((B, S), jnp.int32),
        ),
    )(input_word_ids)


_SC_CHUNK = 128


def _sc_gather_build():
    info = plsc.get_sparse_core_info()
    nw = info.num_cores * info.num_subcores
    rows_per_w = BS // nw
    n_chunks = rows_per_w // _SC_CHUNK

    @functools.partial(
        pl.kernel,
        mesh=plsc.VectorSubcoreMesh(core_axis_name="c", subcore_axis_name="s"),
        out_type=[
            jax.ShapeDtypeStruct((BS, EMB_W), jnp.float32),
            jax.ShapeDtypeStruct((BS, EMB_W), jnp.float32),
        ],
        scratch_types=[
            pltpu.VMEM((rows_per_w,), jnp.int32),
            pltpu.VMEM((rows_per_w,), jnp.int32),
            pltpu.VMEM((_SC_CHUNK, EMB_W), jnp.float32),
            pltpu.VMEM((_SC_CHUNK, EMB_W), jnp.float32),
            pltpu.SemaphoreType.DMA,
            pltpu.SemaphoreType.DMA,
        ],
    )
    def sc_gather(wtab, ptab, wids, pids, we_out, pe_out,
                  widx_v, pidx_v, rows0, rows1, sem0, sem1):
        wid = lax.axis_index("s") * info.num_cores + lax.axis_index("c")
        base = wid * rows_per_w
        pltpu.sync_copy(wids.at[pl.ds(base, rows_per_w)], widx_v)
        pltpu.sync_copy(pids.at[pl.ds(base, rows_per_w)], pidx_v)
        bufs = (rows0, rows1)
        sems = (sem0, sem1)
        jobs = [(wtab, widx_v, we_out, c) for c in range(n_chunks)]
        jobs += [(ptab, pidx_v, pe_out, c) for c in range(n_chunks)]
        copies = []
        for j, (tab, idx_v, out, c) in enumerate(jobs):
            copies.append(pltpu.async_copy(
                tab.at[idx_v.at[pl.ds(c * _SC_CHUNK, _SC_CHUNK)]],
                bufs[j % 2], sems[j % 2]))
            if j >= 1:
                ptab_, pidx_, pout_, pc_ = jobs[j - 1]
                copies[j - 1].wait()
                pltpu.sync_copy(bufs[(j - 1) % 2],
                                pout_.at[pl.ds(base + pc_ * _SC_CHUNK,
                                               _SC_CHUNK)])
        ltab_, lidx_, lout_, lc_ = jobs[-1]
        copies[-1].wait()
        pltpu.sync_copy(bufs[(len(jobs) - 1) % 2],
                        lout_.at[pl.ds(base + lc_ * _SC_CHUNK, _SC_CHUNK)])

    return sc_gather


_RA = 512
_CA = 1024


def _fused_body(seqc_ref, seqr_ref, mask_ref, we_ref, pe_ref, tid_ref,
                temb_ref, g_ref, bt_ref, proj_ref,
                attn_ref, emb_ref):
    x = we_ref[0] + pe_ref[0]
    t = tid_ref[0].astype(jnp.float32)
    t0 = temb_ref[0:1, :]
    t1 = temb_ref[1:2, :]
    x = x + t0 + t * (t1 - t0)
    mean = jnp.mean(x, axis=1, keepdims=True)
    xc = x - mean
    var = jnp.mean(xc * xc, axis=1, keepdims=True)
    y = xc * lax.rsqrt(var + 1e-12) * g_ref[...] + bt_ref[...]
    emb_ref[0] = jnp.dot(y, proj_ref[...], preferred_element_type=jnp.float32)
    sc = seqc_ref[...]
    sr = seqr_ref[...]
    mk = mask_ref[...].astype(jnp.float32)
    for c in range(S // _CA):
        lo, hi = c * _CA, (c + 1) * _CA
        eq = (sc == sr[:, :, lo:hi]).astype(jnp.float32)
        attn_ref[:, :, lo:hi] = eq * mk[:, :, lo:hi]


def _run_fused(seq_ids, input_mask, we, pe, input_type_ids,
               type_emb, ln_gamma, ln_beta, proj_kernel):
    return pl.pallas_call(
        _fused_body,
        grid=(B, S // _RA),
        in_specs=[
            pl.BlockSpec((1, _RA, 1), lambda b, j: (b, j, 0)),
            pl.BlockSpec((1, 1, S), lambda b, j: (b, 0, 0)),
            pl.BlockSpec((1, 1, S), lambda b, j: (b, 0, 0)),
            pl.BlockSpec((1, _RA, EMB_W), lambda b, j: (b, j, 0)),
            pl.BlockSpec((1, _RA, EMB_W), lambda b, j: (b, j, 0)),
            pl.BlockSpec((1, _RA, 1), lambda b, j: (b, j, 0)),
            pl.BlockSpec((2, EMB_W), lambda b, j: (0, 0)),
            pl.BlockSpec((1, EMB_W), lambda b, j: (0, 0)),
            pl.BlockSpec((1, EMB_W), lambda b, j: (0, 0)),
            pl.BlockSpec((EMB_W, HIDDEN), lambda b, j: (0, 0)),
        ],
        out_specs=[
            pl.BlockSpec((1, _RA, S), lambda b, j: (b, j, 0)),
            pl.BlockSpec((1, _RA, HIDDEN), lambda b, j: (b, j, 0)),
        ],
        out_shape=[
            jax.ShapeDtypeStruct((B, S, S), jnp.float32),
            jax.ShapeDtypeStruct((B, S, HIDDEN), jnp.float32),
        ],
    )(seq_ids.reshape(B, S, 1), seq_ids.reshape(B, 1, S),
      input_mask.reshape(B, 1, S), we.reshape(B, S, EMB_W),
      pe.reshape(B, S, EMB_W), input_type_ids.reshape(B, S, 1),
      type_emb, ln_gamma.reshape(1, EMB_W), ln_beta.reshape(1, EMB_W),
      proj_kernel)


def kernel(input_word_ids, input_mask, input_type_ids, word_emb, type_emb,
           pos_emb, ln_gamma, ln_beta, proj_kernel):
    seq_ids, pos_ids = _run_scan(input_word_ids)
    we, pe = _sc_gather_build()(word_emb, pos_emb,
                                input_word_ids.reshape(BS),
                                pos_ids.reshape(BS))
    attn, emb = _run_fused(seq_ids, input_mask, we, pe, input_type_ids,
                           type_emb, ln_gamma, ln_beta, proj_kernel)
    return emb, attn
    we, pe = _sc_gather_build()(
        word_emb, pos_emb,
        input_word_ids.reshape(BS), pos_ids.reshape(BS))
    attn = _run_attn(seq_ids, input_mask)
    emb = _run_emb(we, pe, input_type_ids, type_emb, ln_gamma, ln_beta,
                   proj_kernel)
    return emb.reshape(B, S, HIDDEN), attn


# speedup vs baseline: 1.2593x; 1.0270x over previous
import functools

import jax
import jax.numpy as jnp
from jax import lax
from jax.experimental import pallas as pl
from jax.experimental.pallas import tpu as pltpu
from jax.experimental.pallas import tpu_sc as plsc

B, S = 8, 2048
EMB_W, HIDDEN = 128, 768
BS = B * S


def _scan_body(wid_ref, seq_ref, pos_ref):
    w = wid_ref[...]
    start = (w == w[:, 0:1]).astype(jnp.int32)
    iota = lax.broadcasted_iota(jnp.int32, (B, S), 1)
    m = iota * start
    k = 1
    while k < S:
        m = jnp.maximum(m, jnp.concatenate(
            [jnp.zeros((B, k), jnp.int32), m[:, : S - k]], axis=1))
        k *= 2
    seq_ref[...] = m
    pos_ref[...] = iota - m


def _run_scan(input_word_ids):
    return pl.pallas_call(
        _scan_body,
        out_shape=(
            jax.ShapeDtypeStruct((B, S), jnp.int32),
            jax.ShapeDtypeStruct((B, S), jnp.int32),
        ),
    )(input_word_ids)


_SC_CHUNK = 128


def _sc_gather_build():
    info = plsc.get_sparse_core_info()
    nw = info.num_cores * info.num_subcores
    rows_per_w = BS // nw
    n_chunks = rows_per_w // _SC_CHUNK

    @functools.partial(
        pl.kernel,
        mesh=plsc.VectorSubcoreMesh(core_axis_name="c", subcore_axis_name="s"),
        out_type=[
            jax.ShapeDtypeStruct((BS, EMB_W), jnp.float32),
            jax.ShapeDtypeStruct((BS, EMB_W), jnp.float32),
        ],
        scratch_types=[
            pltpu.VMEM((rows_per_w,), jnp.int32),
            pltpu.VMEM((rows_per_w,), jnp.int32),
            pltpu.VMEM((_SC_CHUNK, EMB_W), jnp.float32),
            pltpu.VMEM((_SC_CHUNK, EMB_W), jnp.float32),
            pltpu.SemaphoreType.DMA,
            pltpu.SemaphoreType.DMA,
        ],
    )
    def sc_gather(wtab, ptab, wids, pids, we_out, pe_out,
                  widx_v, pidx_v, rows0, rows1, sem0, sem1):
        wid = lax.axis_index("s") * info.num_cores + lax.axis_index("c")
        base = wid * rows_per_w
        pltpu.sync_copy(wids.at[pl.ds(base, rows_per_w)], widx_v)
        pltpu.sync_copy(pids.at[pl.ds(base, rows_per_w)], pidx_v)
        bufs = (rows0, rows1)
        sems = (sem0, sem1)
        jobs = [(wtab, widx_v, we_out, c) for c in range(n_chunks)]
        jobs += [(ptab, pidx_v, pe_out, c) for c in range(n_chunks)]
        copies = []
        for j, (tab, idx_v, out, c) in enumerate(jobs):
            copies.append(pltpu.async_copy(
                tab.at[idx_v.at[pl.ds(c * _SC_CHUNK, _SC_CHUNK)]],
                bufs[j % 2], sems[j % 2]))
            if j >= 1:
                ptab_, pidx_, pout_, pc_ = jobs[j - 1]
                copies[j - 1].wait()
                pltpu.sync_copy(bufs[(j - 1) % 2],
                                pout_.at[pl.ds(base + pc_ * _SC_CHUNK,
                                               _SC_CHUNK)])
        ltab_, lidx_, lout_, lc_ = jobs[-1]
        copies[-1].wait()
        pltpu.sync_copy(bufs[(len(jobs) - 1) % 2],
                        lout_.at[pl.ds(base + lc_ * _SC_CHUNK, _SC_CHUNK)])

    return sc_gather


_RA = 1024
_CA = 512


def _fused_body(seqc_ref, seqr_ref, mask_ref, we_ref, pe_ref, tid_ref,
                temb_ref, g_ref, bt_ref, proj_ref,
                attn_ref, emb_ref):
    x = we_ref[0] + pe_ref[0]
    t = tid_ref[0].astype(jnp.float32)
    t0 = temb_ref[0:1, :]
    t1 = temb_ref[1:2, :]
    x = x + t0 + t * (t1 - t0)
    mean = jnp.mean(x, axis=1, keepdims=True)
    xc = x - mean
    var = jnp.mean(xc * xc, axis=1, keepdims=True)
    y = xc * lax.rsqrt(var + 1e-12) * g_ref[...] + bt_ref[...]
    emb_ref[0] = jnp.dot(y, proj_ref[...], preferred_element_type=jnp.float32)
    sc = seqc_ref[...]
    sr = seqr_ref[...]
    mk = mask_ref[...].astype(jnp.float32)
    for c in range(S // _CA):
        lo, hi = c * _CA, (c + 1) * _CA
        eq = (sc == sr[:, :, lo:hi]).astype(jnp.float32)
        attn_ref[:, :, lo:hi] = eq * mk[:, :, lo:hi]


def _run_fused(seq_ids, input_mask, we, pe, input_type_ids,
               type_emb, ln_gamma, ln_beta, proj_kernel):
    return pl.pallas_call(
        _fused_body,
        grid=(B, S // _RA),
        in_specs=[
            pl.BlockSpec((1, _RA, 1), lambda b, j: (b, j, 0)),
            pl.BlockSpec((1, 1, S), lambda b, j: (b, 0, 0)),
            pl.BlockSpec((1, 1, S), lambda b, j: (b, 0, 0)),
            pl.BlockSpec((1, _RA, EMB_W), lambda b, j: (b, j, 0)),
            pl.BlockSpec((1, _RA, EMB_W), lambda b, j: (b, j, 0)),
            pl.BlockSpec((1, _RA, 1), lambda b, j: (b, j, 0)),
            pl.BlockSpec((2, EMB_W), lambda b, j: (0, 0)),
            pl.BlockSpec((1, EMB_W), lambda b, j: (0, 0)),
            pl.BlockSpec((1, EMB_W), lambda b, j: (0, 0)),
            pl.BlockSpec((EMB_W, HIDDEN), lambda b, j: (0, 0)),
        ],
        out_specs=[
            pl.BlockSpec((1, _RA, S), lambda b, j: (b, j, 0)),
            pl.BlockSpec((1, _RA, HIDDEN), lambda b, j: (b, j, 0)),
        ],
        out_shape=[
            jax.ShapeDtypeStruct((B, S, S), jnp.float32),
            jax.ShapeDtypeStruct((B, S, HIDDEN), jnp.float32),
        ],
    )(seq_ids.reshape(B, S, 1), seq_ids.reshape(B, 1, S),
      input_mask.reshape(B, 1, S), we.reshape(B, S, EMB_W),
      pe.reshape(B, S, EMB_W), input_type_ids.reshape(B, S, 1),
      type_emb, ln_gamma.reshape(1, EMB_W), ln_beta.reshape(1, EMB_W),
      proj_kernel)


def kernel(input_word_ids, input_mask, input_type_ids, word_emb, type_emb,
           pos_emb, ln_gamma, ln_beta, proj_kernel):
    seq_ids, pos_ids = _run_scan(input_word_ids)
    we, pe = _sc_gather_build()(word_emb, pos_emb,
                                input_word_ids.reshape(BS),
                                pos_ids.reshape(BS))
    attn, emb = _run_fused(seq_ids, input_mask, we, pe, input_type_ids,
                           type_emb, ln_gamma, ln_beta, proj_kernel)
    return emb, attn
    we, pe = _sc_gather_build()(
        word_emb, pos_emb,
        input_word_ids.reshape(BS), pos_ids.reshape(BS))
    attn = _run_attn(seq_ids, input_mask)
    emb = _run_emb(we, pe, input_type_ids, type_emb, ln_gamma, ln_beta,
                   proj_kernel)
    return emb.reshape(B, S, HIDDEN), attn
